# baseline (speedup 1.0000x reference)
import functools

import jax
import jax.numpy as jnp
from jax.experimental import pallas as pl
from jax.experimental.pallas import tpu as pltpu

TM = 512


def _gcn_body(a_ref, h_ref, w_ref, b_ref, o_ref, hw_ref, *, d, l):
    @pl.when(pl.program_id(1) == 0)
    def _():
        for ll in range(l):
            hw_ref[:, ll * d:(ll + 1) * d] = jax.lax.dot_general(
                h_ref[0, :, ll, :], w_ref[...],
                (((1,), (1,)), ((), ())),
                preferred_element_type=jnp.float32)

    out = jnp.dot(a_ref[0], hw_ref[...], preferred_element_type=jnp.float32)
    for ll in range(l):
        o_ref[0, :, ll, :] = jnp.maximum(
            out[:, ll * d:(ll + 1) * d] + b_ref[...], 0.0)


def kernel(prop_state, A, W, b):
    B, N, L, D = prop_state.shape
    bias = b.reshape(1, D)

    grid = (B, N // TM)
    return pl.pallas_call(
        functools.partial(_gcn_body, d=D, l=L),
        grid=grid,
        in_specs=[
            pl.BlockSpec((1, TM, N), lambda bi, i: (bi, i, 0)),
            pl.BlockSpec((1, N, L, D), lambda bi, i: (bi, 0, 0, 0)),
            pl.BlockSpec((D, D), lambda bi, i: (0, 0)),
            pl.BlockSpec((1, D), lambda bi, i: (0, 0)),
        ],
        out_specs=pl.BlockSpec((1, TM, L, D), lambda bi, i: (bi, i, 0, 0)),
        out_shape=jax.ShapeDtypeStruct((B, N, L, D), jnp.float32),
        scratch_shapes=[pltpu.VMEM((N, L * D), jnp.float32)],
        compiler_params=pltpu.CompilerParams(
            dimension_semantics=("arbitrary", "arbitrary")),
    )(A, prop_state, W, bias)


# speedup vs baseline: 1.2310x; 1.2310x over previous
import functools

import jax
import jax.numpy as jnp
from jax.experimental import pallas as pl
from jax.experimental.pallas import tpu as pltpu

TM = 2048


def _gcn_body(a_ref, h_ref, w_ref, b_ref, o_ref, hw_ref, *, d, l):
    @pl.when(pl.program_id(1) == 0)
    def _():
        for ll in range(l):
            hw_ref[:, ll * d:(ll + 1) * d] = jax.lax.dot_general(
                h_ref[0, :, ll, :], w_ref[...],
                (((1,), (1,)), ((), ())),
                preferred_element_type=jnp.float32)

    out = jnp.dot(a_ref[0], hw_ref[...], preferred_element_type=jnp.float32)
    for ll in range(l):
        o_ref[0, :, ll, :] = jnp.maximum(
            out[:, ll * d:(ll + 1) * d] + b_ref[...], 0.0)


def kernel(prop_state, A, W, b):
    B, N, L, D = prop_state.shape
    bias = b.reshape(1, D)

    grid = (B, N // TM)
    return pl.pallas_call(
        functools.partial(_gcn_body, d=D, l=L),
        grid=grid,
        in_specs=[
            pl.BlockSpec((1, TM, N), lambda bi, i: (bi, i, 0)),
            pl.BlockSpec((1, N, L, D), lambda bi, i: (bi, 0, 0, 0)),
            pl.BlockSpec((D, D), lambda bi, i: (0, 0)),
            pl.BlockSpec((1, D), lambda bi, i: (0, 0)),
        ],
        out_specs=pl.BlockSpec((1, TM, L, D), lambda bi, i: (bi, i, 0, 0)),
        out_shape=jax.ShapeDtypeStruct((B, N, L, D), jnp.float32),
        scratch_shapes=[pltpu.VMEM((N, L * D), jnp.float32)],
        compiler_params=pltpu.CompilerParams(
            dimension_semantics=("arbitrary", "arbitrary")),
    )(A, prop_state, W, bias)


# speedup vs baseline: 1.2410x; 1.0081x over previous
import functools

import jax
import jax.numpy as jnp
from jax.experimental import pallas as pl
from jax.experimental.pallas import tpu as pltpu

TM = 2048


def _gcn_body(a_ref, h_ref, w_ref, b_ref, o_ref, hw_ref, *, d, l):
    for ll in range(l):
        hw_ref[:, ll * d:(ll + 1) * d] = jax.lax.dot_general(
            h_ref[0, :, ll, :], w_ref[...],
            (((1,), (1,)), ((), ())),
            preferred_element_type=jnp.float32)

    out = jnp.dot(a_ref[0], hw_ref[...], preferred_element_type=jnp.float32)
    for ll in range(l):
        o_ref[0, :, ll, :] = jnp.maximum(
            out[:, ll * d:(ll + 1) * d] + b_ref[...], 0.0)


def kernel(prop_state, A, W, b):
    B, N, L, D = prop_state.shape
    bias = b.reshape(1, D)

    grid = (B,)
    return pl.pallas_call(
        functools.partial(_gcn_body, d=D, l=L),
        grid=grid,
        in_specs=[
            pl.BlockSpec((1, TM, N), lambda bi: (bi, 0, 0)),
            pl.BlockSpec((1, N, L, D), lambda bi: (bi, 0, 0, 0)),
            pl.BlockSpec((D, D), lambda bi: (0, 0)),
            pl.BlockSpec((1, D), lambda bi: (0, 0)),
        ],
        out_specs=pl.BlockSpec((1, TM, L, D), lambda bi: (bi, 0, 0, 0)),
        out_shape=jax.ShapeDtypeStruct((B, N, L, D), jnp.float32),
        scratch_shapes=[pltpu.VMEM((N, L * D), jnp.float32)],
        compiler_params=pltpu.CompilerParams(
            dimension_semantics=("arbitrary",)),
    )(A, prop_state, W, bias)
